# baseline (speedup 1.0000x reference)
import jax
import jax.numpy as jnp
from jax.experimental import pallas as pl
from jax.experimental.pallas import tpu as pltpu

_CHUNK = 4000
_NBUF = 3
_ROWS = 20000


def _mlp_pipelined_body(x_hbm, w1_ref, b1_ref, w2_ref, b2_ref, o_hbm,
                        xbuf, obuf, in_sems, out_sems):
    n = x_hbm.shape[0]
    nc = n // _CHUNK

    wc = jnp.dot(w1_ref[...], w2_ref[...],
                 preferred_element_type=jnp.float32).astype(jnp.bfloat16)
    bc = jnp.dot(b1_ref[...], w2_ref[...],
                 preferred_element_type=jnp.float32) + b2_ref[...]

    def in_copy(k, slot):
        return pltpu.make_async_copy(
            x_hbm.at[pl.ds(k * _CHUNK, _CHUNK), :], xbuf.at[slot],
            in_sems.at[slot])

    def out_copy(k, slot):
        return pltpu.make_async_copy(
            obuf.at[slot], o_hbm.at[pl.ds(k * _CHUNK, _CHUNK), :],
            out_sems.at[slot])

    for s in range(_NBUF):
        in_copy(s, s).start()

    def step(k, carry):
        slot = jax.lax.rem(k, _NBUF)
        in_copy(k, slot).wait()

        @pl.when(k >= _NBUF)
        def _drain_out():
            out_copy(k - _NBUF, slot).wait()

        o = jnp.dot(xbuf[slot].astype(jnp.bfloat16), wc,
                    preferred_element_type=jnp.float32)
        obuf[slot] = o + bc
        out_copy(k, slot).start()

        @pl.when(k + _NBUF < nc)
        def _prefetch_in():
            in_copy(k + _NBUF, slot).start()

        return carry

    jax.lax.fori_loop(0, nc, step, 0)

    for t in range(max(nc - _NBUF, 0), nc):
        out_copy(t, t % _NBUF).wait()


def _mlp_grid_body(x_ref, w1_ref, b1_ref, w2_ref, b2_ref, o_ref, wc_ref, bc_ref):
    @pl.when(pl.program_id(0) == 0)
    def _fold_weights():
        wc = jnp.dot(w1_ref[...], w2_ref[...], preferred_element_type=jnp.float32)
        wc_ref[...] = wc.astype(jnp.bfloat16)
        bc_ref[...] = jnp.dot(b1_ref[...], w2_ref[...],
                              preferred_element_type=jnp.float32) + b2_ref[...]

    xb = x_ref[...].astype(jnp.bfloat16)
    o = jnp.dot(xb, wc_ref[...], preferred_element_type=jnp.float32)
    o_ref[...] = o + bc_ref[...]


def kernel(x, graph_x, edge_index, W_ec, b_ec, W1, b1, W2, b2):
    n, d_in = x.shape
    hid = W1.shape[1]
    d_out = W2.shape[1]
    b1r = b1.reshape(1, hid)
    b2r = b2.reshape(1, d_out)

    if n % _CHUNK == 0 and n // _CHUNK >= _NBUF:
        return pl.pallas_call(
            _mlp_pipelined_body,
            in_specs=[
                pl.BlockSpec(memory_space=pl.ANY),
                pl.BlockSpec((d_in, hid), lambda: (0, 0)),
                pl.BlockSpec((1, hid), lambda: (0, 0)),
                pl.BlockSpec((hid, d_out), lambda: (0, 0)),
                pl.BlockSpec((1, d_out), lambda: (0, 0)),
            ],
            out_specs=pl.BlockSpec(memory_space=pl.ANY),
            out_shape=jax.ShapeDtypeStruct((n, d_out), x.dtype),
            scratch_shapes=[
                pltpu.VMEM((_NBUF, _CHUNK, d_in), jnp.float32),
                pltpu.VMEM((_NBUF, _CHUNK, d_out), jnp.float32),
                pltpu.SemaphoreType.DMA((_NBUF,)),
                pltpu.SemaphoreType.DMA((_NBUF,)),
            ],
        )(x, W1, b1r, W2, b2r)

    grid = (pl.cdiv(n, _ROWS),)
    return pl.pallas_call(
        _mlp_grid_body,
        grid=grid,
        in_specs=[
            pl.BlockSpec((_ROWS, d_in), lambda i: (i, 0)),
            pl.BlockSpec((d_in, hid), lambda i: (0, 0)),
            pl.BlockSpec((1, hid), lambda i: (0, 0)),
            pl.BlockSpec((hid, d_out), lambda i: (0, 0)),
            pl.BlockSpec((1, d_out), lambda i: (0, 0)),
        ],
        out_specs=pl.BlockSpec((_ROWS, d_out), lambda i: (i, 0)),
        out_shape=jax.ShapeDtypeStruct((n, d_out), x.dtype),
        scratch_shapes=[
            pltpu.VMEM((d_in, d_out), jnp.bfloat16),
            pltpu.VMEM((1, d_out), jnp.float32),
        ],
    )(x, W1, b1r, W2, b2r)


# speedup vs baseline: 1.0308x; 1.0308x over previous
import jax
import jax.numpy as jnp
from jax.experimental import pallas as pl
from jax.experimental.pallas import tpu as pltpu

_CHUNK = 10000
_NBUF = 3
_ROWS = 20000


def _mlp_pipelined_body(x_hbm, w1_ref, b1_ref, w2_ref, b2_ref, o_hbm,
                        xbuf, obuf, in_sems, out_sems):
    n = x_hbm.shape[0]
    nc = n // _CHUNK

    wc = jnp.dot(w1_ref[...], w2_ref[...],
                 preferred_element_type=jnp.float32).astype(jnp.bfloat16)
    bc = jnp.dot(b1_ref[...], w2_ref[...],
                 preferred_element_type=jnp.float32) + b2_ref[...]

    def in_copy(k, slot):
        return pltpu.make_async_copy(
            x_hbm.at[pl.ds(k * _CHUNK, _CHUNK), :], xbuf.at[slot],
            in_sems.at[slot])

    def out_copy(k, slot):
        return pltpu.make_async_copy(
            obuf.at[slot], o_hbm.at[pl.ds(k * _CHUNK, _CHUNK), :],
            out_sems.at[slot])

    for s in range(_NBUF):
        in_copy(s, s).start()

    def step(k, carry):
        slot = jax.lax.rem(k, _NBUF)
        in_copy(k, slot).wait()

        @pl.when(k >= _NBUF)
        def _drain_out():
            out_copy(k - _NBUF, slot).wait()

        o = jnp.dot(xbuf[slot].astype(jnp.bfloat16), wc,
                    preferred_element_type=jnp.float32)
        obuf[slot] = o + bc
        out_copy(k, slot).start()

        @pl.when(k + _NBUF < nc)
        def _prefetch_in():
            in_copy(k + _NBUF, slot).start()

        return carry

    jax.lax.fori_loop(0, nc, step, 0)

    for t in range(max(nc - _NBUF, 0), nc):
        out_copy(t, t % _NBUF).wait()


def _mlp_grid_body(x_ref, w1_ref, b1_ref, w2_ref, b2_ref, o_ref, wc_ref, bc_ref):
    @pl.when(pl.program_id(0) == 0)
    def _fold_weights():
        wc = jnp.dot(w1_ref[...], w2_ref[...], preferred_element_type=jnp.float32)
        wc_ref[...] = wc.astype(jnp.bfloat16)
        bc_ref[...] = jnp.dot(b1_ref[...], w2_ref[...],
                              preferred_element_type=jnp.float32) + b2_ref[...]

    xb = x_ref[...].astype(jnp.bfloat16)
    o = jnp.dot(xb, wc_ref[...], preferred_element_type=jnp.float32)
    o_ref[...] = o + bc_ref[...]


def kernel(x, graph_x, edge_index, W_ec, b_ec, W1, b1, W2, b2):
    n, d_in = x.shape
    hid = W1.shape[1]
    d_out = W2.shape[1]
    b1r = b1.reshape(1, hid)
    b2r = b2.reshape(1, d_out)

    if n % _CHUNK == 0 and n // _CHUNK >= _NBUF:
        return pl.pallas_call(
            _mlp_pipelined_body,
            in_specs=[
                pl.BlockSpec(memory_space=pl.ANY),
                pl.BlockSpec((d_in, hid), lambda: (0, 0)),
                pl.BlockSpec((1, hid), lambda: (0, 0)),
                pl.BlockSpec((hid, d_out), lambda: (0, 0)),
                pl.BlockSpec((1, d_out), lambda: (0, 0)),
            ],
            out_specs=pl.BlockSpec(memory_space=pl.ANY),
            out_shape=jax.ShapeDtypeStruct((n, d_out), x.dtype),
            scratch_shapes=[
                pltpu.VMEM((_NBUF, _CHUNK, d_in), jnp.float32),
                pltpu.VMEM((_NBUF, _CHUNK, d_out), jnp.float32),
                pltpu.SemaphoreType.DMA((_NBUF,)),
                pltpu.SemaphoreType.DMA((_NBUF,)),
            ],
        )(x, W1, b1r, W2, b2r)

    grid = (pl.cdiv(n, _ROWS),)
    return pl.pallas_call(
        _mlp_grid_body,
        grid=grid,
        in_specs=[
            pl.BlockSpec((_ROWS, d_in), lambda i: (i, 0)),
            pl.BlockSpec((d_in, hid), lambda i: (0, 0)),
            pl.BlockSpec((1, hid), lambda i: (0, 0)),
            pl.BlockSpec((hid, d_out), lambda i: (0, 0)),
            pl.BlockSpec((1, d_out), lambda i: (0, 0)),
        ],
        out_specs=pl.BlockSpec((_ROWS, d_out), lambda i: (i, 0)),
        out_shape=jax.ShapeDtypeStruct((n, d_out), x.dtype),
        scratch_shapes=[
            pltpu.VMEM((d_in, d_out), jnp.bfloat16),
            pltpu.VMEM((1, d_out), jnp.float32),
        ],
    )(x, W1, b1r, W2, b2r)
